# Initial kernel scaffold; baseline (speedup 1.0000x reference)
import jax
import jax.numpy as jnp
from jax.experimental import pallas as pl


def kernel(x, edge_index, batch, W0, b0, W1, b1, Wout, bout):
    raise NotImplementedError("write your pallas kernel here")


# speedup vs baseline: 1.8700x; 1.8700x over previous
import functools

import jax
import jax.numpy as jnp
from jax import lax
from jax.experimental import pallas as pl
from jax.experimental.pallas import tpu as pltpu
from jax.experimental.pallas import tpu_sc as plsc

N = 10000
E = 160000
D = 256
H = 512
G = 64
KHOP = 3

CF = 128
DW = 128
NPAD = 10240
ECH = 128
NW = 32
EPAD = 163840
EPT = EPAD // NW
NITER = EPT // ECH
RPS = NPAD // 16
ZR = RPS // 4

BN = 400
NBLK = N // BN

_sc_mesh = plsc.VectorSubcoreMesh(core_axis_name="c", subcore_axis_name="s")


@functools.partial(
    pl.kernel,
    out_type=jax.ShapeDtypeStruct((2 * NPAD, DW), jnp.float32),
    mesh=_sc_mesh,
    scratch_types=[
        pltpu.VMEM((1, ECH), jnp.int32),
        pltpu.VMEM((ECH, DW), jnp.float32),
        pltpu.VMEM((ZR, DW), jnp.float32),
        pltpu.VMEM_SHARED((NPAD, DW), jnp.float32),
    ],
)
def _deg_kernel(col_hbm, out_hbm, cidx_v, ones_v, zero_v, acc_sh):
    cid = lax.axis_index("c")
    sid = lax.axis_index("s")
    w = sid * 2 + cid

    @pl.loop(0, ECH)
    def _(r):
        for j in range(DW // 16):
            ones_v[r, pl.ds(j * 16, 16)] = jnp.full((16,), 1.0, jnp.float32)

    @pl.loop(0, ZR)
    def _(r):
        for j in range(DW // 16):
            zero_v[r, pl.ds(j * 16, 16)] = jnp.zeros((16,), jnp.float32)

    @pl.loop(0, RPS // ZR)
    def _(b):
        pltpu.sync_copy(zero_v, acc_sh.at[pl.ds(sid * RPS + b * ZR, ZR)])

    plsc.subcore_barrier()

    @pl.loop(0, NITER)
    def _(t):
        off = w * EPT + t * ECH
        pltpu.sync_copy(col_hbm.at[pl.ds(off, ECH)], cidx_v.at[0])
        pltpu.sync_copy(ones_v, acc_sh.at[cidx_v.at[0]], add=True)

    plsc.subcore_barrier()
    pltpu.sync_copy(acc_sh.at[pl.ds(sid * RPS, RPS)],
                    out_hbm.at[pl.ds(cid * NPAD + sid * RPS, RPS)])


@functools.partial(
    pl.kernel,
    out_type=jax.ShapeDtypeStruct((2 * NPAD, CF), jnp.float32),
    mesh=_sc_mesh,
    scratch_types=[
        pltpu.VMEM((1, ECH), jnp.int32),
        pltpu.VMEM((1, ECH), jnp.int32),
        pltpu.VMEM((ECH, CF), jnp.float32),
        pltpu.VMEM((ZR, CF), jnp.float32),
        pltpu.VMEM_SHARED((NPAD, CF), jnp.float32),
        pltpu.SemaphoreType.DMA,
    ],
)
def _prop_kernel(u_hbm, row_hbm, col_hbm, out_hbm,
                 ridx_v, cidx_v, rows_v, zero_v, acc_sh, sem):
    cid = lax.axis_index("c")
    sid = lax.axis_index("s")
    w = sid * 2 + cid

    @pl.loop(0, ZR)
    def _(r):
        for j in range(CF // 16):
            zero_v[r, pl.ds(j * 16, 16)] = jnp.zeros((16,), jnp.float32)

    @pl.loop(0, RPS // ZR)
    def _(b):
        pltpu.sync_copy(zero_v, acc_sh.at[pl.ds(sid * RPS + b * ZR, ZR)])

    plsc.subcore_barrier()

    @pl.loop(0, NITER)
    def _(t):
        off = w * EPT + t * ECH
        pltpu.sync_copy(row_hbm.at[pl.ds(off, ECH)], ridx_v.at[0])
        pltpu.sync_copy(col_hbm.at[pl.ds(off, ECH)], cidx_v.at[0])
        pltpu.async_copy(u_hbm.at[ridx_v.at[0]], rows_v, sem).wait()
        pltpu.sync_copy(rows_v, acc_sh.at[cidx_v.at[0]], add=True)

    plsc.subcore_barrier()
    pltpu.sync_copy(acc_sh.at[pl.ds(sid * RPS, RPS)],
                    out_hbm.at[pl.ds(cid * NPAD + sid * RPS, RPS)])


def _dinv_from(degp_ref):
    degs = degp_ref[0] + degp_ref[1]
    deg = degs[:, :1]
    return jnp.where(deg > 0, lax.rsqrt(deg), 0.0)


def _prep_body(degp_ref, x_ref, w_ref, out_ref, u0_ref, u1_ref):
    dinv = _dinv_from(degp_ref)
    x = x_ref[...]
    out_ref[...] = jnp.dot(x, w_ref[...], preferred_element_type=jnp.float32, precision=lax.Precision.HIGHEST)
    u = dinv * x
    u0_ref[...] = u[:, :CF]
    u1_ref[...] = u[:, CF:]


def _merge_body(nf, degp_ref, outp_ref, w_ref, *rest):
    s_refs = rest[:nf]
    out_ref = rest[nf]
    u_refs = rest[nf + 1:]
    dinv = _dinv_from(degp_ref)
    ssum = jnp.concatenate([s[0] + s[1] for s in s_refs], axis=1)
    h = dinv * ssum
    out_ref[...] = outp_ref[...] + jnp.dot(
        h, w_ref[...], preferred_element_type=jnp.float32, precision=lax.Precision.HIGHEST)
    for j in range(nf):
        u_refs[j][...] = dinv * h[:, j * CF:(j + 1) * CF]


def _final1_body(degp_ref, outp_ref, w3_ref, b_ref, w10_ref, s0_ref, s1_ref,
                 out2_ref, u0_ref, u1_ref, u2_ref, u3_ref):
    dinv = _dinv_from(degp_ref)
    ssum = jnp.concatenate([s0_ref[0] + s0_ref[1], s1_ref[0] + s1_ref[1]],
                           axis=1)
    h = dinv * ssum
    y = outp_ref[...] + jnp.dot(
        h, w3_ref[...], preferred_element_type=jnp.float32, precision=lax.Precision.HIGHEST) + b_ref[...]
    x2 = jnp.where(y >= 0, y, 0.25 * y)
    out2_ref[...] = jnp.dot(x2, w10_ref[...],
                            preferred_element_type=jnp.float32, precision=lax.Precision.HIGHEST)
    u = dinv * x2
    for j, ur in enumerate((u0_ref, u1_ref, u2_ref, u3_ref)):
        ur[...] = u[:, j * CF:(j + 1) * CF]


def _final2_body(degp_ref, outp_ref, w3_ref, b_ref, wout_ref, bout_ref,
                 bat_ref, s0_ref, s1_ref, s2_ref, s3_ref, res_ref):
    i = pl.program_id(0)
    dinv = _dinv_from(degp_ref)
    ssum = jnp.concatenate(
        [s[0] + s[1] for s in (s0_ref, s1_ref, s2_ref, s3_ref)], axis=1)
    h = dinv * ssum
    y = outp_ref[...] + jnp.dot(
        h, w3_ref[...], preferred_element_type=jnp.float32, precision=lax.Precision.HIGHEST) + b_ref[...]
    x3 = jnp.where(y >= 0, y, 0.25 * y)
    z = jnp.dot(x3, wout_ref[...], preferred_element_type=jnp.float32, precision=lax.Precision.HIGHEST)
    ids = bat_ref[0]
    gid = lax.broadcasted_iota(jnp.int32, (G, BN), 0)
    mask = (gid == ids).astype(jnp.float32)
    part = jnp.dot(mask, z, preferred_element_type=jnp.float32, precision=lax.Precision.HIGHEST)

    @pl.when(i == 0)
    def _():
        res_ref[...] = jnp.broadcast_to(bout_ref[...], (G, 1))

    res_ref[...] += part


def _sblock(i):
    return (0, i, 0)


_DEG_SPEC = pl.BlockSpec((2, BN, DW), _sblock)
_S_SPEC = pl.BlockSpec((2, BN, CF), _sblock)
_H_SPEC = pl.BlockSpec((BN, H), lambda i: (i, 0))
_U_SPEC = pl.BlockSpec((BN, CF), lambda i: (i, 0))
_H_OUT = jax.ShapeDtypeStruct((N, H), jnp.float32)
_U_OUT = jax.ShapeDtypeStruct((N, CF), jnp.float32)


def _full(shape):
    return pl.BlockSpec(shape, lambda i: tuple(0 for _ in shape))


_prep = pl.pallas_call(
    _prep_body,
    grid=(NBLK,),
    in_specs=[_DEG_SPEC, pl.BlockSpec((BN, D), lambda i: (i, 0)),
              _full((D, H))],
    out_specs=[_H_SPEC, _U_SPEC, _U_SPEC],
    out_shape=[_H_OUT, _U_OUT, _U_OUT],
)

_merge2 = pl.pallas_call(
    functools.partial(_merge_body, 2),
    grid=(NBLK,),
    in_specs=[_DEG_SPEC, _H_SPEC, _full((D, H)), _S_SPEC, _S_SPEC],
    out_specs=[_H_SPEC, _U_SPEC, _U_SPEC],
    out_shape=[_H_OUT, _U_OUT, _U_OUT],
)

_merge4 = pl.pallas_call(
    functools.partial(_merge_body, 4),
    grid=(NBLK,),
    in_specs=[_DEG_SPEC, _H_SPEC, _full((H, H)),
              _S_SPEC, _S_SPEC, _S_SPEC, _S_SPEC],
    out_specs=[_H_SPEC, _U_SPEC, _U_SPEC, _U_SPEC, _U_SPEC],
    out_shape=[_H_OUT, _U_OUT, _U_OUT, _U_OUT, _U_OUT],
)

_final1 = pl.pallas_call(
    _final1_body,
    grid=(NBLK,),
    in_specs=[_DEG_SPEC, _H_SPEC, _full((D, H)), _full((1, H)),
              _full((H, H)), _S_SPEC, _S_SPEC],
    out_specs=[_H_SPEC, _U_SPEC, _U_SPEC, _U_SPEC, _U_SPEC],
    out_shape=[_H_OUT, _U_OUT, _U_OUT, _U_OUT, _U_OUT],
)

_final2 = pl.pallas_call(
    _final2_body,
    grid=(NBLK,),
    in_specs=[_DEG_SPEC, _H_SPEC, _full((H, H)), _full((1, H)),
              _full((H, 1)), _full((1, 1)),
              pl.BlockSpec((1, 1, BN), lambda i: (i, 0, 0)),
              _S_SPEC, _S_SPEC, _S_SPEC, _S_SPEC],
    out_specs=pl.BlockSpec((G, 1), lambda i: (0, 0)),
    out_shape=jax.ShapeDtypeStruct((G, 1), jnp.float32),
)


def _prop(u, row_p, col_p):
    return _prop_kernel(u, row_p, col_p).reshape(2, NPAD, CF)


def kernel(x, edge_index, batch, W0, b0, W1, b1, Wout, bout):
    row = edge_index[0]
    col = edge_index[1]
    pad = EPAD - E
    row_p = jnp.concatenate([row, jnp.zeros((pad,), jnp.int32)])
    col_p = jnp.concatenate([col, jnp.full((pad,), N, jnp.int32)])
    batch2d = batch.reshape(NBLK, 1, BN)
    b0r = b0.reshape(1, H)
    b1r = b1.reshape(1, H)
    boutr = bout.reshape(1, 1)

    degp = _deg_kernel(col_p).reshape(2, NPAD, DW)

    out, u0, u1 = _prep(degp, x, W0[0])
    for k in (1, 2):
        s0 = _prop(u0, row_p, col_p)
        s1 = _prop(u1, row_p, col_p)
        out, u0, u1 = _merge2(degp, out, W0[k], s0, s1)
    s0 = _prop(u0, row_p, col_p)
    s1 = _prop(u1, row_p, col_p)
    out2, v0, v1, v2, v3 = _final1(degp, out, W0[3], b0r, W1[0], s0, s1)

    for k in (1, 2):
        t0 = _prop(v0, row_p, col_p)
        t1 = _prop(v1, row_p, col_p)
        t2 = _prop(v2, row_p, col_p)
        t3 = _prop(v3, row_p, col_p)
        out2, v0, v1, v2, v3 = _merge4(degp, out2, W1[k], t0, t1, t2, t3)
    t0 = _prop(v0, row_p, col_p)
    t1 = _prop(v1, row_p, col_p)
    t2 = _prop(v2, row_p, col_p)
    t3 = _prop(v3, row_p, col_p)

    return _final2(degp, out2, W1[3], b1r, Wout, boutr, batch2d,
                   t0, t1, t2, t3)


# speedup vs baseline: 2.3948x; 1.2806x over previous
import functools

import jax
import jax.numpy as jnp
from jax import lax
from jax.experimental import pallas as pl
from jax.experimental.pallas import tpu as pltpu
from jax.experimental.pallas import tpu_sc as plsc

N = 10000
E = 160000
D = 256
H = 512
G = 64
KHOP = 3

CF = 128
DW = 128
NPAD = 10240
ECH = 128
NW = 32
EPAD = 163840
EPT = EPAD // NW
NITER = EPT // ECH
RPS = NPAD // 16
ZR = RPS // 4

BN = 400
NBLK = N // BN

_sc_mesh = plsc.VectorSubcoreMesh(core_axis_name="c", subcore_axis_name="s")


@functools.partial(
    pl.kernel,
    out_type=jax.ShapeDtypeStruct((2 * NPAD, DW), jnp.float32),
    mesh=_sc_mesh,
    scratch_types=[
        pltpu.VMEM((NITER, ECH), jnp.int32),
        pltpu.VMEM((ECH, DW), jnp.float32),
        pltpu.VMEM((ECH, DW), jnp.float32),
        pltpu.VMEM_SHARED((NPAD, DW), jnp.float32),
        pltpu.SemaphoreType.DMA,
    ],
)
def _deg_kernel(col_hbm, out_hbm, cidx_all, ones_v, zero_v, acc_sh, sems):
    cid = lax.axis_index("c")
    sid = lax.axis_index("s")
    w = sid * 2 + cid

    pltpu.sync_copy(col_hbm.at[pl.ds(w * NITER, NITER)], cidx_all)

    @pl.loop(0, ECH)
    def _(r):
        for j in range(DW // 16):
            ones_v[r, pl.ds(j * 16, 16)] = jnp.full((16,), 1.0, jnp.float32)

    @pl.loop(0, ECH)
    def _(r):
        for j in range(DW // 16):
            zero_v[r, pl.ds(j * 16, 16)] = jnp.zeros((16,), jnp.float32)

    @pl.loop(0, RPS // ECH)
    def _(b):
        pltpu.sync_copy(zero_v, acc_sh.at[pl.ds(sid * RPS + b * ECH, ECH)])

    plsc.subcore_barrier()

    descs = [pltpu.async_copy(ones_v, acc_sh.at[cidx_all.at[t]], sems,
                              add=True) for t in range(NITER)]
    for d in descs:
        d.wait()

    plsc.subcore_barrier()
    pltpu.sync_copy(acc_sh.at[pl.ds(sid * RPS, RPS)],
                    out_hbm.at[pl.ds(cid * NPAD + sid * RPS, RPS)])


@functools.partial(
    pl.kernel,
    out_type=jax.ShapeDtypeStruct((2 * NPAD, CF), jnp.float32),
    mesh=_sc_mesh,
    scratch_types=[
        pltpu.VMEM((NITER, ECH), jnp.int32),
        pltpu.VMEM((NITER, ECH), jnp.int32),
        pltpu.VMEM((ECH, CF), jnp.float32),
        pltpu.VMEM((ECH, CF), jnp.float32),
        pltpu.VMEM_SHARED((NPAD, CF), jnp.float32),
        pltpu.SemaphoreType.DMA,
        pltpu.SemaphoreType.DMA,
    ],
)
def _prop_kernel(u_hbm, row_hbm, col_hbm, out_hbm,
                 ridx_all, cidx_all, buf0, buf1,
                 acc_sh, semg, sems):
    cid = lax.axis_index("c")
    sid = lax.axis_index("s")
    w = sid * 2 + cid
    bufs = (buf0, buf1)

    pltpu.sync_copy(row_hbm.at[pl.ds(w * NITER, NITER)], ridx_all)
    pltpu.sync_copy(col_hbm.at[pl.ds(w * NITER, NITER)], cidx_all)

    @pl.loop(0, ECH)
    def _(r):
        for j in range(CF // 16):
            buf0[r, pl.ds(j * 16, 16)] = jnp.zeros((16,), jnp.float32)

    for q in range(RPS // ECH):
        pltpu.sync_copy(buf0, acc_sh.at[pl.ds(sid * RPS + q * ECH, ECH)])
    plsc.subcore_barrier()

    def gather(t):
        return pltpu.async_copy(u_hbm.at[ridx_all.at[t]], bufs[t % 2], semg)

    def scatter(t):
        return pltpu.async_copy(bufs[t % 2], acc_sh.at[cidx_all.at[t]],
                                sems, add=True)

    dg_cur = gather(0)
    ds_prev = None
    for t in range(NITER):
        dg_cur.wait()
        if ds_prev is not None:
            ds_prev.wait()
        if t + 1 < NITER:
            dg_cur = gather(t + 1)
        ds_prev = scatter(t)
    ds_prev.wait()

    plsc.subcore_barrier()
    pltpu.sync_copy(acc_sh.at[pl.ds(sid * RPS, RPS)],
                    out_hbm.at[pl.ds(cid * NPAD + sid * RPS, RPS)])


def _dinv_from(degp_ref):
    degs = degp_ref[0] + degp_ref[1]
    deg = degs[:, :1]
    return jnp.where(deg > 0, lax.rsqrt(deg), 0.0)


def _prep_body(degp_ref, x_ref, w_ref, out_ref, u0_ref, u1_ref):
    dinv = _dinv_from(degp_ref)
    x = x_ref[...]
    out_ref[...] = jnp.dot(x, w_ref[...], preferred_element_type=jnp.float32, precision=lax.Precision.HIGHEST)
    u = dinv * x
    u0_ref[...] = u[:, :CF]
    u1_ref[...] = u[:, CF:]


def _merge_body(nf, degp_ref, outp_ref, w_ref, *rest):
    s_refs = rest[:nf]
    out_ref = rest[nf]
    u_refs = rest[nf + 1:]
    dinv = _dinv_from(degp_ref)
    ssum = jnp.concatenate([s[0] + s[1] for s in s_refs], axis=1)
    h = dinv * ssum
    out_ref[...] = outp_ref[...] + jnp.dot(
        h, w_ref[...], preferred_element_type=jnp.float32, precision=lax.Precision.HIGHEST)
    for j in range(nf):
        u_refs[j][...] = dinv * h[:, j * CF:(j + 1) * CF]


def _final1_body(degp_ref, outp_ref, w3_ref, b_ref, w10_ref, s0_ref, s1_ref,
                 out2_ref, u0_ref, u1_ref, u2_ref, u3_ref):
    dinv = _dinv_from(degp_ref)
    ssum = jnp.concatenate([s0_ref[0] + s0_ref[1], s1_ref[0] + s1_ref[1]],
                           axis=1)
    h = dinv * ssum
    y = outp_ref[...] + jnp.dot(
        h, w3_ref[...], preferred_element_type=jnp.float32, precision=lax.Precision.HIGHEST) + b_ref[...]
    x2 = jnp.where(y >= 0, y, 0.25 * y)
    out2_ref[...] = jnp.dot(x2, w10_ref[...],
                            preferred_element_type=jnp.float32, precision=lax.Precision.HIGHEST)
    u = dinv * x2
    for j, ur in enumerate((u0_ref, u1_ref, u2_ref, u3_ref)):
        ur[...] = u[:, j * CF:(j + 1) * CF]


def _final2_body(degp_ref, outp_ref, w3_ref, b_ref, wout_ref, bout_ref,
                 bat_ref, s0_ref, s1_ref, s2_ref, s3_ref, res_ref):
    i = pl.program_id(0)
    dinv = _dinv_from(degp_ref)
    ssum = jnp.concatenate(
        [s[0] + s[1] for s in (s0_ref, s1_ref, s2_ref, s3_ref)], axis=1)
    h = dinv * ssum
    y = outp_ref[...] + jnp.dot(
        h, w3_ref[...], preferred_element_type=jnp.float32, precision=lax.Precision.HIGHEST) + b_ref[...]
    x3 = jnp.where(y >= 0, y, 0.25 * y)
    z = jnp.dot(x3, wout_ref[...], preferred_element_type=jnp.float32, precision=lax.Precision.HIGHEST)
    ids = bat_ref[0]
    gid = lax.broadcasted_iota(jnp.int32, (G, BN), 0)
    mask = (gid == ids).astype(jnp.float32)
    part = jnp.dot(mask, z, preferred_element_type=jnp.float32, precision=lax.Precision.HIGHEST)

    @pl.when(i == 0)
    def _():
        res_ref[...] = jnp.broadcast_to(bout_ref[...], (G, 1))

    res_ref[...] += part


def _sblock(i):
    return (0, i, 0)


_DEG_SPEC = pl.BlockSpec((2, BN, DW), _sblock)
_S_SPEC = pl.BlockSpec((2, BN, CF), _sblock)
_H_SPEC = pl.BlockSpec((BN, H), lambda i: (i, 0))
_U_SPEC = pl.BlockSpec((BN, CF), lambda i: (i, 0))
_H_OUT = jax.ShapeDtypeStruct((N, H), jnp.float32)
_U_OUT = jax.ShapeDtypeStruct((N, CF), jnp.float32)


def _full(shape):
    return pl.BlockSpec(shape, lambda i: tuple(0 for _ in shape))


_prep = pl.pallas_call(
    _prep_body,
    grid=(NBLK,),
    in_specs=[_DEG_SPEC, pl.BlockSpec((BN, D), lambda i: (i, 0)),
              _full((D, H))],
    out_specs=[_H_SPEC, _U_SPEC, _U_SPEC],
    out_shape=[_H_OUT, _U_OUT, _U_OUT],
)

_merge2 = pl.pallas_call(
    functools.partial(_merge_body, 2),
    grid=(NBLK,),
    in_specs=[_DEG_SPEC, _H_SPEC, _full((D, H)), _S_SPEC, _S_SPEC],
    out_specs=[_H_SPEC, _U_SPEC, _U_SPEC],
    out_shape=[_H_OUT, _U_OUT, _U_OUT],
)

_merge4 = pl.pallas_call(
    functools.partial(_merge_body, 4),
    grid=(NBLK,),
    in_specs=[_DEG_SPEC, _H_SPEC, _full((H, H)),
              _S_SPEC, _S_SPEC, _S_SPEC, _S_SPEC],
    out_specs=[_H_SPEC, _U_SPEC, _U_SPEC, _U_SPEC, _U_SPEC],
    out_shape=[_H_OUT, _U_OUT, _U_OUT, _U_OUT, _U_OUT],
)

_final1 = pl.pallas_call(
    _final1_body,
    grid=(NBLK,),
    in_specs=[_DEG_SPEC, _H_SPEC, _full((D, H)), _full((1, H)),
              _full((H, H)), _S_SPEC, _S_SPEC],
    out_specs=[_H_SPEC, _U_SPEC, _U_SPEC, _U_SPEC, _U_SPEC],
    out_shape=[_H_OUT, _U_OUT, _U_OUT, _U_OUT, _U_OUT],
)

_final2 = pl.pallas_call(
    _final2_body,
    grid=(NBLK,),
    in_specs=[_DEG_SPEC, _H_SPEC, _full((H, H)), _full((1, H)),
              _full((H, 1)), _full((1, 1)),
              pl.BlockSpec((1, 1, BN), lambda i: (i, 0, 0)),
              _S_SPEC, _S_SPEC, _S_SPEC, _S_SPEC],
    out_specs=pl.BlockSpec((G, 1), lambda i: (0, 0)),
    out_shape=jax.ShapeDtypeStruct((G, 1), jnp.float32),
)


def _prop(u, row_p, col_p):
    return _prop_kernel(u, row_p, col_p).reshape(2, NPAD, CF)


def kernel(x, edge_index, batch, W0, b0, W1, b1, Wout, bout):
    row = edge_index[0]
    col = edge_index[1]
    pad = EPAD - E
    row_p = jnp.concatenate(
        [row, jnp.zeros((pad,), jnp.int32)]).reshape(NW * NITER, ECH)
    col_p = jnp.concatenate(
        [col, jnp.full((pad,), N, jnp.int32)]).reshape(NW * NITER, ECH)
    batch2d = batch.reshape(NBLK, 1, BN)
    b0r = b0.reshape(1, H)
    b1r = b1.reshape(1, H)
    boutr = bout.reshape(1, 1)

    degp = _deg_kernel(col_p).reshape(2, NPAD, DW)

    out, u0, u1 = _prep(degp, x, W0[0])
    for k in (1, 2):
        s0 = _prop(u0, row_p, col_p)
        s1 = _prop(u1, row_p, col_p)
        out, u0, u1 = _merge2(degp, out, W0[k], s0, s1)
    s0 = _prop(u0, row_p, col_p)
    s1 = _prop(u1, row_p, col_p)
    out2, v0, v1, v2, v3 = _final1(degp, out, W0[3], b0r, W1[0], s0, s1)

    for k in (1, 2):
        t0 = _prop(v0, row_p, col_p)
        t1 = _prop(v1, row_p, col_p)
        t2 = _prop(v2, row_p, col_p)
        t3 = _prop(v3, row_p, col_p)
        out2, v0, v1, v2, v3 = _merge4(degp, out2, W1[k], t0, t1, t2, t3)
    t0 = _prop(v0, row_p, col_p)
    t1 = _prop(v1, row_p, col_p)
    t2 = _prop(v2, row_p, col_p)
    t3 = _prop(v3, row_p, col_p)

    return _final2(degp, out2, W1[3], b1r, Wout, boutr, batch2d,
                   t0, t1, t2, t3)
